# baseline (speedup 1.0000x reference)
import functools

import jax
import jax.numpy as jnp
from jax import lax
from jax.experimental import pallas as pl
from jax.experimental.pallas import tpu as pltpu
from jax.experimental.pallas import tpu_sc as plsc

CHUNK = 2


@functools.lru_cache(maxsize=None)
def _build(B, L, EMB):
    info = plsc.get_sparse_core_info()
    NC, NS = info.num_cores, info.num_subcores
    NW = NC * NS
    RPW = B // NW
    NCH = RPW // CHUNK
    HL = L // 2
    W = 2 * EMB

    @functools.partial(
        pl.kernel,
        mesh=plsc.VectorSubcoreMesh(core_axis_name="c", subcore_axis_name="s"),
        compiler_params=pltpu.CompilerParams(use_tc_tiling_on_sc=False),
        out_type=jax.ShapeDtypeStruct((B * L // 2, W), jnp.float32),
        scratch_types=[
            pltpu.VMEM((CHUNK, L), jnp.int32),
            pltpu.VMEM((CHUNK, L), jnp.int32),
            pltpu.VMEM((CHUNK * L, W), jnp.float32),
            pltpu.VMEM((CHUNK * L, W), jnp.float32),
            pltpu.VMEM((L, EMB), jnp.float32),
            pltpu.SemaphoreType.DMA,
            pltpu.SemaphoreType.DMA,
            pltpu.SemaphoreType.DMA,
        ],
    )
    def k(x_hbm, emb_hbm, pos_hbm, out_hbm, ib0, ib1, rb0, rb1, pos_v,
          gsem, osem, isem):
        wid = lax.axis_index("s") * NC + lax.axis_index("c")
        base = wid * RPW
        ibufs = (ib0, ib1)
        rbufs = (rb0, rb1)

        def idx_start(c, ib):
            pltpu.async_copy(x_hbm.at[pl.ds(base + c * CHUNK, CHUNK)], ib, isem)

        def idx_wait():
            pltpu.make_async_copy(
                x_hbm.at[pl.ds(base, CHUNK)], ib0, isem).wait()

        def gather_start(ib, rb):
            for r in range(CHUNK):
                pltpu.async_copy(
                    emb_hbm.at[ib.at[r]], rb.at[pl.ds(r * L, L)], gsem)

        def gather_wait():
            for r in range(CHUNK):
                pltpu.make_async_copy(
                    emb_hbm.at[ib0.at[r]], rb0.at[pl.ds(r * L, L)],
                    gsem).wait()

        def out_start(c, rb):
            pltpu.async_copy(
                rb.at[pl.ds(0, CHUNK * HL)],
                out_hbm.at[pl.ds((base + c * CHUNK) * HL, CHUNK * HL)],
                osem)

        def out_wait():
            pltpu.make_async_copy(
                rb0.at[pl.ds(0, CHUNK * HL)],
                out_hbm.at[pl.ds(base * HL, CHUNK * HL)], osem).wait()

        def vadd_pack(rb):
            for r in range(CHUNK):
                def add_j(j, carry, r=r):
                    s0 = 2 * r * HL + 2 * j
                    d = r * HL + j
                    for q in range(EMB // 16):
                        sl = pl.ds(q * 16, 16)
                        sh = pl.ds(EMB + q * 16, 16)
                        a = rb[s0, sl] + pos_v[2 * j, sl]
                        b = rb[s0 + 1, sl] + pos_v[2 * j + 1, sl]
                        rb[d, sl] = a
                        rb[d, sh] = b
                    return carry
                lax.fori_loop(0, HL, add_j, 0)

        def step(c, A):
            idx_wait()
            out_wait()
            gather_start(ibufs[1 - A], rbufs[1 - A])
            gather_wait()
            idx_start(c + 2, ibufs[A])
            vadd_pack(rbufs[A])
            out_start(c, rbufs[A])

        pltpu.sync_copy(pos_hbm.at[pl.ds(0, L)], pos_v)
        pltpu.sync_copy(x_hbm.at[pl.ds(base, CHUNK)], ib0)
        gather_start(ib0, rb0)
        idx_start(1, ib1)

        idx_wait()
        gather_start(ib1, rb1)
        gather_wait()
        idx_start(2, ib0)
        vadd_pack(rb0)
        out_start(0, rb0)

        def pair(cp, carry):
            c = 2 * cp
            step(c - 1, 1)
            step(c, 0)
            return carry
        lax.fori_loop(1, NCH // 2 - 1, pair, 0)

        step(NCH - 3, 1)
        idx_wait()
        out_wait()
        gather_start(ibufs[1], rbufs[1])
        gather_wait()
        vadd_pack(rb0)
        out_start(NCH - 2, rb0)
        out_wait()
        gather_wait()
        vadd_pack(rb1)
        out_start(NCH - 1, rb1)
        out_wait()

    return k


def kernel(x, emb_table, pos_table):
    B, L = x.shape
    V, EMB = emb_table.shape
    emb_dup = jnp.concatenate([emb_table, emb_table], axis=1)
    k = _build(B, L, EMB)
    out = k(x.astype(jnp.int32), emb_dup, pos_table)
    o = out.reshape(B, L, EMB)
    ot = lax.optimization_barrier(jnp.transpose(o, (1, 2, 0)))
    return jnp.transpose(ot, (2, 0, 1))


# speedup vs baseline: 1.2228x; 1.2228x over previous
import functools

import jax
import jax.numpy as jnp
from jax import lax
from jax.experimental import pallas as pl
from jax.experimental.pallas import tpu as pltpu
from jax.experimental.pallas import tpu_sc as plsc

CHUNK = 4


@functools.lru_cache(maxsize=None)
def _build(B, L, EMB):
    info = plsc.get_sparse_core_info()
    NC, NS = info.num_cores, info.num_subcores
    NW = NC * NS
    RPW = B // NW
    NCH = RPW // CHUNK

    @functools.partial(
        pl.kernel,
        mesh=plsc.VectorSubcoreMesh(core_axis_name="c", subcore_axis_name="s"),
        compiler_params=pltpu.CompilerParams(use_tc_tiling_on_sc=False),
        out_type=jax.ShapeDtypeStruct((B, L, EMB), jnp.float32),
        scratch_types=[
            pltpu.VMEM((CHUNK, L), jnp.int32),
            pltpu.VMEM((CHUNK, L), jnp.int32),
            pltpu.VMEM((CHUNK, L, EMB), jnp.float32),
            pltpu.VMEM((CHUNK, L, EMB), jnp.float32),
            pltpu.VMEM((L, EMB), jnp.float32),
            pltpu.SemaphoreType.DMA,
            pltpu.SemaphoreType.DMA,
            pltpu.SemaphoreType.DMA,
        ],
    )
    def k(x_hbm, emb_hbm, pos_hbm, out_hbm, ib0, ib1, rb0, rb1, pos_v,
          gsem, osem, isem):
        wid = lax.axis_index("s") * NC + lax.axis_index("c")
        base = wid * RPW
        ibufs = (ib0, ib1)
        rbufs = (rb0, rb1)

        def idx_start(c, ib):
            pltpu.async_copy(x_hbm.at[pl.ds(base + c * CHUNK, CHUNK)], ib, isem)

        def idx_wait():
            pltpu.make_async_copy(
                x_hbm.at[pl.ds(base, CHUNK)], ib0, isem).wait()

        def gather_start(ib, rb):
            for r in range(CHUNK):
                pltpu.async_copy(emb_hbm.at[ib.at[r]], rb.at[r], gsem)

        def gather_wait():
            for r in range(CHUNK):
                pltpu.make_async_copy(
                    emb_hbm.at[ib0.at[r]], rb0.at[r], gsem).wait()

        def out_start(c, rb):
            pltpu.async_copy(
                rb, out_hbm.at[pl.ds(base + c * CHUNK, CHUNK)], osem)

        def out_wait():
            pltpu.make_async_copy(
                rb0, out_hbm.at[pl.ds(base, CHUNK)], osem).wait()

        def vadd(rb):
            def add_i(i, carry):
                for j in range(EMB // 16):
                    sl = pl.ds(j * 16, 16)
                    p = pos_v[i, sl]
                    for r in range(CHUNK):
                        rb[r, i, sl] = rb[r, i, sl] + p
                return carry
            lax.fori_loop(0, L, add_i, 0)

        pltpu.sync_copy(pos_hbm.at[pl.ds(0, L)], pos_v)
        pltpu.sync_copy(x_hbm.at[pl.ds(base, CHUNK)], ib0)
        gather_start(ib0, rb0)
        idx_start(1, ib1)

        for c in range(NCH):
            A = c & 1
            if c + 1 < NCH:
                idx_wait()
                if c >= 1:
                    out_wait()
                gather_start(ibufs[1 - A], rbufs[1 - A])
            elif c >= 1:
                out_wait()
            gather_wait()
            if c + 2 < NCH:
                idx_start(c + 2, ibufs[A])
            vadd(rbufs[A])
            out_start(c, rbufs[A])
        out_wait()

    return k


def kernel(x, emb_table, pos_table):
    B, L = x.shape
    EMB = emb_table.shape[1]
    k = _build(B, L, EMB)
    return k(x.astype(jnp.int32), emb_table, pos_table)
